# baseline (speedup 1.0000x reference)
import functools

import jax
import jax.numpy as jnp
from jax import lax
from jax.experimental import pallas as pl
from jax.experimental.pallas import tpu as pltpu

N = 10000
E = 160000
H = 256
HALF = 128
LAYERS = 6

BN = 2000
BE = 4000
NB = N // BN
EB = E // BE

_f32 = jnp.float32


def _dot(a, b):
    return jnp.dot(a, b, preferred_element_type=_f32,
                   precision=lax.Precision.HIGHEST)


def _enc_body(xm, te, wn1, bn1, wn2, wt1, bt1, wt2, b2, out):
    h1 = jnp.maximum(_dot(xm[...], wn1[...]) + bn1[...], 0.0)
    h2 = jnp.maximum(_dot(te[...], wt1[...]) + bt1[...], 0.0)
    out[...] = _dot(h1, wn2[...]) + _dot(h2, wt2[...]) + b2[...]


def _edge_enc_body(ef, w1, b1, out):
    out[...] = jnp.maximum(_dot(ef[...], w1[...]) + b1[...], 0.0)


def _ce_body(re_ref, m_ref, out):
    out[...] = _dot(re_ref[...], m_ref[...])


def _ab_body(h_ref, w_ref, b_ref, out):
    out[...] = _dot(h_ref[...], w_ref[0]) + b_ref[0]


def _node_body(h_ref, slo, shi, cnt_ref, u1h, kk, kvec, b1u, u2, b2u, out):
    s = jnp.concatenate([slo[...], shi[...]], axis=1)
    pre = (_dot(h_ref[...], u1h[...]) + _dot(s, kk[...])
           + cnt_ref[...] * kvec[...] + b1u[...])
    out[...] = h_ref[...] + _dot(jnp.maximum(pre, 0.0), u2[...]) + b2u[...]


def _dec_body(h_ref, d1, b1, d2, b2, out):
    hh = jnp.maximum(_dot(h_ref[...], d1[...]) + b1[...], 0.0)
    out[...] = _dot(hh, d2[...]) + b2[...]


def _full(shape):
    return pl.BlockSpec(shape, lambda *_: tuple(0 for _ in shape))


def _encode_nodes(xm, te, p_node, p_time):
    b2 = (p_node["b2"] + p_time["b2"]).reshape(1, H)
    return pl.pallas_call(
        _enc_body,
        grid=(NB,),
        in_specs=[
            pl.BlockSpec((BN, 8), lambda j: (j, 0)),
            pl.BlockSpec((BN, 128), lambda j: (j, 0)),
            _full((8, H)), _full((1, H)), _full((H, H)),
            _full((128, H)), _full((1, H)), _full((H, H)), _full((1, H)),
        ],
        out_specs=pl.BlockSpec((BN, H), lambda j: (j, 0)),
        out_shape=jax.ShapeDtypeStruct((N, H), _f32),
    )(xm, te,
      jnp.pad(p_node["W1"], ((0, 1), (0, 0))), p_node["b1"].reshape(1, H),
      p_node["W2"],
      p_time["W1"], p_time["b1"].reshape(1, H), p_time["W2"], b2)


def _encode_edges(ef8, p_edge):
    return pl.pallas_call(
        _edge_enc_body,
        grid=(EB,),
        in_specs=[
            pl.BlockSpec((BE, 8), lambda j: (j, 0)),
            _full((8, H)), _full((1, H)),
        ],
        out_specs=pl.BlockSpec((BE, H), lambda j: (j, 0)),
        out_shape=jax.ShapeDtypeStruct((E, H), _f32),
    )(ef8, jnp.pad(p_edge["W1"], ((0, 4), (0, 0))), p_edge["b1"].reshape(1, H))


def _ce(relu_e, m):
    return pl.pallas_call(
        _ce_body,
        grid=(EB, 2),
        in_specs=[
            pl.BlockSpec((BE, H), lambda j, c: (j, 0)),
            pl.BlockSpec((H, HALF), lambda j, c: (0, c)),
        ],
        out_specs=pl.BlockSpec((BE, HALF), lambda j, c: (c * EB + j, 0)),
        out_shape=jax.ShapeDtypeStruct((2 * E, HALF), _f32),
    )(relu_e, m)


def _ab(h, ws, wd, bias):
    wst = jnp.stack([ws, wd])
    bst = jnp.stack([jnp.zeros((1, H), _f32), bias.reshape(1, H)])
    return pl.pallas_call(
        _ab_body,
        grid=(NB, 2, 2),
        in_specs=[
            pl.BlockSpec((BN, H), lambda j, a, c: (j, 0)),
            pl.BlockSpec((1, H, HALF), lambda j, a, c: (a, 0, c)),
            pl.BlockSpec((1, 1, HALF), lambda j, a, c: (a, 0, c)),
        ],
        out_specs=pl.BlockSpec((BN, HALF), lambda j, a, c: ((a * 2 + c) * NB + j, 0)),
        out_shape=jax.ShapeDtypeStruct((4 * N, HALF), _f32),
    )(h, wst, bst)


def _node_update(h, s2, cnt2, u1h, kk, kvec, b1u, u2, b2u):
    return pl.pallas_call(
        _node_body,
        grid=(NB,),
        in_specs=[
            pl.BlockSpec((BN, H), lambda j: (j, 0)),
            pl.BlockSpec((BN, HALF), lambda j: (j, 0)),
            pl.BlockSpec((BN, HALF), lambda j: (NB + j, 0)),
            pl.BlockSpec((BN, 1), lambda j: (j, 0)),
            _full((H, H)), _full((H, H)), _full((1, H)), _full((1, H)),
            _full((H, H)), _full((1, H)),
        ],
        out_specs=pl.BlockSpec((BN, H), lambda j: (j, 0)),
        out_shape=jax.ShapeDtypeStruct((N, H), _f32),
    )(h, s2, s2, cnt2, u1h, kk, kvec.reshape(1, H), b1u.reshape(1, H),
      u2, b2u.reshape(1, H))


def _decode(h, p_dec):
    d2 = jnp.pad(p_dec["W2"], ((0, 0), (0, 128 - p_dec["W2"].shape[1])))
    b2 = jnp.pad(p_dec["b2"], (0, 128 - p_dec["b2"].shape[0])).reshape(1, 128)
    out = pl.pallas_call(
        _dec_body,
        grid=(NB,),
        in_specs=[
            pl.BlockSpec((BN, H), lambda j: (j, 0)),
            _full((H, H)), _full((1, H)), _full((H, 128)), _full((1, 128)),
        ],
        out_specs=pl.BlockSpec((BN, 128), lambda j: (j, 0)),
        out_shape=jax.ShapeDtypeStruct((N, 128), _f32),
    )(h, p_dec["W1"], p_dec["b1"].reshape(1, H), d2, b2)
    return out[:, :3]


def _edge_pass(t_tab, ce2, src, dst):
    a = jnp.concatenate([t_tab[:N], t_tab[N:2 * N]], axis=1)
    b = jnp.concatenate([t_tab[2 * N:3 * N], t_tab[3 * N:]], axis=1)
    ce = jnp.concatenate([ce2[:E], ce2[E:]], axis=1)
    r = jnp.maximum(a[src] + b[dst] + ce, 0.0)
    s = jax.ops.segment_sum(r, dst, num_segments=N)
    return jnp.concatenate([s[:, :HALF], s[:, HALF:]], axis=0)


def _edge_counts(dst):
    return jax.ops.segment_sum(jnp.ones((E,), _f32), dst, num_segments=N)


def kernel(x_t, node_meta, edge_index, edge_feats, t_embed, batch_index, params):
    del batch_index
    src = edge_index[0].astype(jnp.int32)
    dst = edge_index[1].astype(jnp.int32)

    xm = jnp.pad(jnp.concatenate([x_t, node_meta], axis=1), ((0, 0), (0, 1)))
    ef8 = jnp.pad(edge_feats, ((0, 0), (0, 4)))

    h = _encode_nodes(xm, t_embed, params["enc_node"], params["enc_time"])
    relu_e = _encode_edges(ef8, params["enc_edge"])

    ee = params["enc_edge"]
    cnt2 = _edge_counts(dst).reshape(N, 1)

    for blk in params["blocks"]:
        w1 = blk["msg"]["W1"]
        w1s, w1d, w1e = w1[:H], w1[H:2 * H], w1[2 * H:]
        m = jnp.dot(ee["W2"], w1e, precision=lax.Precision.HIGHEST)
        bias = blk["msg"]["b1"] + jnp.dot(ee["b2"], w1e,
                                          precision=lax.Precision.HIGHEST)
        u1 = blk["node"]["W1"]
        u1h, u1a = u1[:H], u1[H:]
        kk = jnp.dot(blk["msg"]["W2"], u1a, precision=lax.Precision.HIGHEST)
        kvec = jnp.dot(blk["msg"]["b2"], u1a, precision=lax.Precision.HIGHEST)

        ce2 = _ce(relu_e, m)
        t_tab = _ab(h, w1s, w1d, bias)
        s2 = _edge_pass(t_tab, ce2, src, dst)
        h = _node_update(h, s2, cnt2, u1h, kk, kvec, blk["node"]["b1"],
                         blk["node"]["W2"], blk["node"]["b2"])

    return _decode(h, params["dec_node"])


# speedup vs baseline: 3.0765x; 3.0765x over previous
import functools

import jax
import jax.numpy as jnp
from jax import lax
from jax.experimental import pallas as pl
from jax.experimental.pallas import tpu as pltpu
from jax.experimental.pallas import tpu_sc as plsc

N = 10000
E = 160000
H = 256
HALF = 128
LAYERS = 6

BN = 2000
BE = 4000
NB = N // BN
EB = E // BE

_f32 = jnp.float32
_bf16 = jnp.bfloat16


def _hdot(a, b):
    return jnp.dot(a, b, preferred_element_type=_f32,
                   precision=lax.Precision.HIGHEST)


def _bdot(a, b):
    return jnp.dot(a.astype(_bf16), b.astype(_bf16),
                   preferred_element_type=_f32)


def _irne(x):
    u = lax.bitcast_convert_type(x, jnp.int32)
    u = u + jnp.int32(0x7FFF) + ((u >> 16) & jnp.int32(1))
    return lax.bitcast_convert_type(u & jnp.int32(-65536), _f32)


def _enc_body(xm, te, wn1, bn1, wn2, wt1, bt1, wt2, b2, out):
    h1 = jnp.maximum(_bdot(xm[...], wn1[...]) + bn1[...], 0.0)
    h2 = jnp.maximum(_bdot(te[...], wt1[...]) + bt1[...], 0.0)
    out[...] = _bdot(h1, wn2[...]) + _bdot(h2, wt2[...]) + b2[...]


def _edge_enc_body(ef, w1, b1, w2, b2, out):
    hh = jnp.maximum(_bdot(ef[...], w1[...]) + b1[...], 0.0)
    out[...] = (_bdot(hh, w2[...]) + b2[...]).astype(_bf16)


def _ce_body(he_ref, m_ref, out):
    out[...] = jnp.dot(he_ref[...], m_ref[...].astype(_bf16),
                       preferred_element_type=_f32)


def _ab_body(h_ref, w_ref, b_ref, out):
    out[...] = _bdot(h_ref[...], w_ref[0]) + b_ref[0]


def _node_body(h_ref, slo, shi, cnt_ref, u1h, w2r, b2m, b1u, u1a, u2, b2u, out):
    s = jnp.concatenate([slo[...], shi[...]], axis=1)
    agg = _hdot(s, w2r[...]) + cnt_ref[...] * b2m[...]
    pre = (_bdot(h_ref[...], u1h[...]) + _bdot(agg, u1a[...]) + b1u[...])
    out[...] = h_ref[...] + _bdot(jnp.maximum(pre, 0.0), u2[...]) + b2u[...]


def _dec_body(h_ref, d1, b1, d2, b2, out):
    hh = jnp.maximum(_bdot(h_ref[...], d1[...]) + b1[...], 0.0)
    out[...] = _bdot(hh, d2[...]) + b2[...]


def _full(shape):
    return pl.BlockSpec(shape, lambda *_: tuple(0 for _ in shape))


def _encode_nodes(xm, te, p_node, p_time):
    b2 = (p_node["b2"] + p_time["b2"]).reshape(1, H)
    return pl.pallas_call(
        _enc_body,
        grid=(NB,),
        in_specs=[
            pl.BlockSpec((BN, 8), lambda j: (j, 0)),
            pl.BlockSpec((BN, 128), lambda j: (j, 0)),
            _full((8, H)), _full((1, H)), _full((H, H)),
            _full((128, H)), _full((1, H)), _full((H, H)), _full((1, H)),
        ],
        out_specs=pl.BlockSpec((BN, H), lambda j: (j, 0)),
        out_shape=jax.ShapeDtypeStruct((N, H), _f32),
    )(xm, te,
      jnp.pad(p_node["W1"], ((0, 1), (0, 0))), p_node["b1"].reshape(1, H),
      p_node["W2"],
      p_time["W1"], p_time["b1"].reshape(1, H), p_time["W2"], b2)


def _encode_edges(ef8, p_edge):
    return pl.pallas_call(
        _edge_enc_body,
        grid=(EB,),
        in_specs=[
            pl.BlockSpec((BE, 8), lambda j: (j, 0)),
            _full((8, H)), _full((1, H)), _full((H, H)), _full((1, H)),
        ],
        out_specs=pl.BlockSpec((BE, H), lambda j: (j, 0)),
        out_shape=jax.ShapeDtypeStruct((E, H), _bf16),
    )(ef8, jnp.pad(p_edge["W1"], ((0, 4), (0, 0))), p_edge["b1"].reshape(1, H),
      p_edge["W2"], p_edge["b2"].reshape(1, H))


def _ce(relu_e, m):
    return pl.pallas_call(
        _ce_body,
        grid=(EB, 2),
        in_specs=[
            pl.BlockSpec((BE, H), lambda j, c: (j, 0)),
            pl.BlockSpec((H, HALF), lambda j, c: (0, c)),
        ],
        out_specs=pl.BlockSpec((BE, HALF), lambda j, c: (c * EB + j, 0)),
        out_shape=jax.ShapeDtypeStruct((2 * E, HALF), _f32),
    )(relu_e, m)


def _ab(h, ws, wd, bias):
    wst = jnp.stack([ws, wd])
    bst = jnp.stack([jnp.zeros((1, H), _f32), bias.reshape(1, H)])
    return pl.pallas_call(
        _ab_body,
        grid=(NB, 2, 2),
        in_specs=[
            pl.BlockSpec((BN, H), lambda j, a, c: (j, 0)),
            pl.BlockSpec((1, H, HALF), lambda j, a, c: (a, 0, c)),
            pl.BlockSpec((1, 1, HALF), lambda j, a, c: (a, 0, c)),
        ],
        out_specs=pl.BlockSpec((BN, HALF), lambda j, a, c: ((a * 2 + c) * NB + j, 0)),
        out_shape=jax.ShapeDtypeStruct((4 * N, HALF), _f32),
    )(h, wst, bst)


def _node_update(h, s2, cnt2, u1h, w2r, b2m, b1u, u1a, u2, b2u):
    return pl.pallas_call(
        _node_body,
        grid=(NB,),
        in_specs=[
            pl.BlockSpec((BN, H), lambda j: (j, 0)),
            pl.BlockSpec((BN, HALF), lambda j: (j, 0)),
            pl.BlockSpec((BN, HALF), lambda j: (NB + j, 0)),
            pl.BlockSpec((BN, 1), lambda j: (j, 0)),
            _full((H, H)), _full((H, H)), _full((1, H)), _full((1, H)),
            _full((H, H)), _full((H, H)), _full((1, H)),
        ],
        out_specs=pl.BlockSpec((BN, H), lambda j: (j, 0)),
        out_shape=jax.ShapeDtypeStruct((N, H), _f32),
    )(h, s2, s2, cnt2, u1h, w2r, b2m.reshape(1, H), b1u.reshape(1, H),
      u1a, u2, b2u.reshape(1, H))


def _decode(h, p_dec):
    d2 = jnp.pad(p_dec["W2"], ((0, 0), (0, 128 - p_dec["W2"].shape[1])))
    b2 = jnp.pad(p_dec["b2"], (0, 128 - p_dec["b2"].shape[0])).reshape(1, 128)
    out = pl.pallas_call(
        _dec_body,
        grid=(NB,),
        in_specs=[
            pl.BlockSpec((BN, H), lambda j: (j, 0)),
            _full((H, H)), _full((1, H)), _full((H, 128)), _full((1, 128)),
        ],
        out_specs=pl.BlockSpec((BN, 128), lambda j: (j, 0)),
        out_shape=jax.ShapeDtypeStruct((N, 128), _f32),
    )(h, p_dec["W1"], p_dec["b1"].reshape(1, H), d2, b2)
    return out[:, :3]


ROWS = E // 128
RPT = ROWS // 16

_sc_mesh = plsc.VectorSubcoreMesh(core_axis_name="c", subcore_axis_name="s")


def _sc_edge_body(t_hbm, ce_hbm, src_hbm, dst_hbm, out_hbm,
                  gia, gib, dstb, bufa, bufb, bufc, s_sh, sema, semb):
    c = lax.axis_index("c")
    s = lax.axis_index("s")
    zero16 = jnp.zeros((16,), _f32)

    def _zr(r, _):
        for j in range(8):
            bufa[r, pl.ds(j * 16, 16)] = zero16
        return 0
    lax.fori_loop(0, 104, _zr, 0)
    for r in range(6):
        pltpu.sync_copy(bufa.at[pl.ds(0, 104)],
                        s_sh.at[pl.ds(s * 624 + r * 104, 104)])

    @pl.when(s < 2)
    def _():
        pltpu.sync_copy(bufa.at[pl.ds(0, 8)],
                        s_sh.at[pl.ds(9984 + s * 8, 8)])
    plsc.subcore_barrier()

    off_a = jnp.full((16,), c * N, jnp.int32)
    off_b = jnp.full((16,), 2 * N + c * N, jnp.int32)
    base_row = s * RPT

    def _chunk(row0):
        pltpu.sync_copy(src_hbm.at[pl.ds(row0, 1)], gia)
        pltpu.sync_copy(dst_hbm.at[pl.ds(row0, 1)], dstb)
        for j in range(8):
            sl = pl.ds(j * 16, 16)
            gia[0, sl] = gia[0, sl] + off_a
            gib[0, sl] = dstb[0, sl] + off_b
        cpa = pltpu.async_copy(t_hbm.at[gia.at[0]], bufa, sema)
        cpb = pltpu.async_copy(t_hbm.at[gib.at[0]], bufb, semb)
        pltpu.sync_copy(ce_hbm.at[pl.ds(c * E + row0 * 128, 128)], bufc)
        cpa.wait()
        cpb.wait()

        def _rb(r, _):
            for j in range(8):
                sl = pl.ds(j * 16, 16)
                bufc[r, sl] = _irne(jnp.maximum(
                    bufa[r, sl] + bufb[r, sl] + bufc[r, sl], 0.0))
            return 0
        lax.fori_loop(0, 128, _rb, 0)
        pltpu.sync_copy(bufc, s_sh.at[dstb.at[0]], add=True)

    def _main(i, _):
        _chunk(base_row + i)
        return 0
    lax.fori_loop(0, RPT, _main, 0)

    @pl.when(s < 2)
    def _():
        _chunk(16 * RPT + s)

    plsc.subcore_barrier()
    pltpu.sync_copy(s_sh.at[pl.ds(s * 624, 624)],
                    out_hbm.at[pl.ds(c * N + s * 624, 624)])

    @pl.when(s < 2)
    def _():
        pltpu.sync_copy(s_sh.at[pl.ds(9984 + s * 8, 8)],
                        out_hbm.at[pl.ds(c * N + 9984 + s * 8, 8)])


def _edge_pass(t_tab, ce2, src2d, dst2d):
    return pl.kernel(
        _sc_edge_body,
        out_type=jax.ShapeDtypeStruct((2 * N, HALF), _f32),
        mesh=_sc_mesh,
        scratch_types=[
            pltpu.VMEM((1, 128), jnp.int32),
            pltpu.VMEM((1, 128), jnp.int32),
            pltpu.VMEM((1, 128), jnp.int32),
            pltpu.VMEM((128, HALF), _f32),
            pltpu.VMEM((128, HALF), _f32),
            pltpu.VMEM((128, HALF), _f32),
            pltpu.VMEM_SHARED((N, HALF), _f32),
            pltpu.SemaphoreType.DMA,
            pltpu.SemaphoreType.DMA,
        ],
    )(t_tab, ce2, src2d, dst2d)


def _sc_cnt_body(dst_hbm, out_hbm, idx1, vals, cnt_sh, onesb):
    c = lax.axis_index("c")
    s = lax.axis_index("s")
    def _z(i, _):
        vals[pl.ds(i * 16, 16)] = jnp.zeros((16,), _f32)
        return 0
    lax.fori_loop(0, 125, _z, 0)

    @pl.when(s < 5)
    def _():
        pltpu.sync_copy(vals, cnt_sh.at[pl.ds(s * 2000, 2000)])
    plsc.subcore_barrier()

    def _o(i, _):
        onesb[pl.ds(i * 16, 16)] = jnp.ones((16,), _f32)
        return 0
    lax.fori_loop(0, 8, _o, 0)

    wid = c * 16 + s

    def _r(r, _):
        pltpu.sync_copy(dst_hbm.at[pl.ds(wid * 39 + r, 1)], idx1)
        pltpu.sync_copy(onesb, cnt_sh.at[idx1.at[0]], add=True)
        return 0
    lax.fori_loop(0, 39, _r, 0)

    @pl.when(jnp.logical_and(c == 0, s < 2))
    def _():
        pltpu.sync_copy(dst_hbm.at[pl.ds(39 * 32 + s, 1)], idx1)
        pltpu.sync_copy(onesb, cnt_sh.at[idx1.at[0]], add=True)
    plsc.subcore_barrier()

    @pl.when(s < 5)
    def _():
        pltpu.sync_copy(cnt_sh.at[pl.ds(s * 2000, 2000)], vals)
        pltpu.sync_copy(vals, out_hbm.at[pl.ds(c * N + s * 2000, 2000)])


def _edge_counts(dst2d):
    parts = pl.kernel(
        _sc_cnt_body,
        out_type=jax.ShapeDtypeStruct((2 * N,), _f32),
        mesh=_sc_mesh,
        scratch_types=[
            pltpu.VMEM((1, 128), jnp.int32),
            pltpu.VMEM((2000,), _f32),
            pltpu.VMEM_SHARED((N,), _f32),
            pltpu.VMEM((128,), _f32),
        ],
    )(dst2d)
    return parts[:N] + parts[N:]


def kernel(x_t, node_meta, edge_index, edge_feats, t_embed, batch_index, params):
    del batch_index
    src2d = edge_index[0].astype(jnp.int32).reshape(ROWS, 128)
    dst2d = edge_index[1].astype(jnp.int32).reshape(ROWS, 128)

    xm = jnp.pad(jnp.concatenate([x_t, node_meta], axis=1), ((0, 0), (0, 1)))
    ef8 = jnp.pad(edge_feats, ((0, 0), (0, 4)))

    h = _encode_nodes(xm, t_embed, params["enc_node"], params["enc_time"])
    h_edge16 = _encode_edges(ef8, params["enc_edge"])

    cnt2 = _edge_counts(dst2d).reshape(N, 1)

    for blk in params["blocks"]:
        w1 = blk["msg"]["W1"]
        w1s, w1d, w1e = w1[:H], w1[H:2 * H], w1[2 * H:]
        u1 = blk["node"]["W1"]
        u1h, u1a = u1[:H], u1[H:]
        w2r = _irne(blk["msg"]["W2"])

        ce2 = _ce(h_edge16, w1e)
        t_tab = _ab(h, w1s, w1d, blk["msg"]["b1"])
        s2 = _edge_pass(t_tab, ce2, src2d, dst2d)
        h = _node_update(h, s2, cnt2, u1h, w2r, blk["msg"]["b2"],
                         blk["node"]["b1"], u1a, blk["node"]["W2"],
                         blk["node"]["b2"])

    return _decode(h, params["dec_node"])


# speedup vs baseline: 3.4476x; 1.1206x over previous
import functools

import jax
import jax.numpy as jnp
from jax import lax
from jax.experimental import pallas as pl
from jax.experimental.pallas import tpu as pltpu
from jax.experimental.pallas import tpu_sc as plsc

N = 10000
E = 160000
H = 256
HALF = 128
LAYERS = 6

BN = 2000
BE = 4000
NB = N // BN
EB = E // BE

_f32 = jnp.float32
_bf16 = jnp.bfloat16


def _hdot(a, b):
    return jnp.dot(a, b, preferred_element_type=_f32,
                   precision=lax.Precision.HIGHEST)


def _bdot(a, b):
    return jnp.dot(a.astype(_bf16), b.astype(_bf16),
                   preferred_element_type=_f32)


def _irne(x):
    u = lax.bitcast_convert_type(x, jnp.int32)
    u = u + jnp.int32(0x7FFF) + ((u >> 16) & jnp.int32(1))
    return lax.bitcast_convert_type(u & jnp.int32(-65536), _f32)


def _enc_body(xm, te, wn1, bn1, wn2, wt1, bt1, wt2, b2, out):
    h1 = jnp.maximum(_bdot(xm[...], wn1[...]) + bn1[...], 0.0)
    h2 = jnp.maximum(_bdot(te[...], wt1[...]) + bt1[...], 0.0)
    out[...] = _bdot(h1, wn2[...]) + _bdot(h2, wt2[...]) + b2[...]


def _edge_enc_body(ef, w1, b1, w2, b2, out):
    hh = jnp.maximum(_bdot(ef[...], w1[...]) + b1[...], 0.0)
    out[...] = (_bdot(hh, w2[...]) + b2[...]).astype(_bf16)


def _ce_body(he_ref, m_ref, out):
    out[...] = jnp.dot(he_ref[...], m_ref[...].astype(_bf16),
                       preferred_element_type=_f32)


def _ab_body(h_ref, w_ref, b_ref, out):
    out[...] = _bdot(h_ref[...], w_ref[0]) + b_ref[0]


def _node_body(h_ref, slo, shi, cnt_ref, u1h, w2r, b2m, b1u, u1a, u2, b2u, out):
    s = jnp.concatenate([slo[...], shi[...]], axis=1)
    agg = _hdot(s, w2r[...]) + cnt_ref[...] * b2m[...]
    pre = (_bdot(h_ref[...], u1h[...]) + _bdot(agg, u1a[...]) + b1u[...])
    out[...] = h_ref[...] + _bdot(jnp.maximum(pre, 0.0), u2[...]) + b2u[...]


def _dec_body(h_ref, d1, b1, d2, b2, out):
    hh = jnp.maximum(_bdot(h_ref[...], d1[...]) + b1[...], 0.0)
    out[...] = _bdot(hh, d2[...]) + b2[...]


def _full(shape):
    return pl.BlockSpec(shape, lambda *_: tuple(0 for _ in shape))


def _encode_nodes(xm, te, p_node, p_time):
    b2 = (p_node["b2"] + p_time["b2"]).reshape(1, H)
    return pl.pallas_call(
        _enc_body,
        grid=(NB,),
        in_specs=[
            pl.BlockSpec((BN, 8), lambda j: (j, 0)),
            pl.BlockSpec((BN, 128), lambda j: (j, 0)),
            _full((8, H)), _full((1, H)), _full((H, H)),
            _full((128, H)), _full((1, H)), _full((H, H)), _full((1, H)),
        ],
        out_specs=pl.BlockSpec((BN, H), lambda j: (j, 0)),
        out_shape=jax.ShapeDtypeStruct((N, H), _f32),
    )(xm, te,
      jnp.pad(p_node["W1"], ((0, 1), (0, 0))), p_node["b1"].reshape(1, H),
      p_node["W2"],
      p_time["W1"], p_time["b1"].reshape(1, H), p_time["W2"], b2)


def _encode_edges(ef8, p_edge):
    return pl.pallas_call(
        _edge_enc_body,
        grid=(EB,),
        in_specs=[
            pl.BlockSpec((BE, 8), lambda j: (j, 0)),
            _full((8, H)), _full((1, H)), _full((H, H)), _full((1, H)),
        ],
        out_specs=pl.BlockSpec((BE, H), lambda j: (j, 0)),
        out_shape=jax.ShapeDtypeStruct((E, H), _bf16),
    )(ef8, jnp.pad(p_edge["W1"], ((0, 4), (0, 0))), p_edge["b1"].reshape(1, H),
      p_edge["W2"], p_edge["b2"].reshape(1, H))


def _ce(relu_e, m):
    return pl.pallas_call(
        _ce_body,
        grid=(EB, 2),
        in_specs=[
            pl.BlockSpec((BE, H), lambda j, c: (j, 0)),
            pl.BlockSpec((H, HALF), lambda j, c: (0, c)),
        ],
        out_specs=pl.BlockSpec((BE, HALF), lambda j, c: (c * EB + j, 0)),
        out_shape=jax.ShapeDtypeStruct((2 * E, HALF), _f32),
    )(relu_e, m)


def _ab(h, ws, wd, bias):
    wst = jnp.stack([ws, wd])
    bst = jnp.stack([jnp.zeros((1, H), _f32), bias.reshape(1, H)])
    return pl.pallas_call(
        _ab_body,
        grid=(NB, 2, 2),
        in_specs=[
            pl.BlockSpec((BN, H), lambda j, a, c: (j, 0)),
            pl.BlockSpec((1, H, HALF), lambda j, a, c: (a, 0, c)),
            pl.BlockSpec((1, 1, HALF), lambda j, a, c: (a, 0, c)),
        ],
        out_specs=pl.BlockSpec((BN, HALF), lambda j, a, c: ((a * 2 + c) * NB + j, 0)),
        out_shape=jax.ShapeDtypeStruct((4 * N, HALF), _f32),
    )(h, wst, bst)


def _node_update(h, s2, cnt2, u1h, w2r, b2m, b1u, u1a, u2, b2u):
    return pl.pallas_call(
        _node_body,
        grid=(NB,),
        in_specs=[
            pl.BlockSpec((BN, H), lambda j: (j, 0)),
            pl.BlockSpec((BN, HALF), lambda j: (j, 0)),
            pl.BlockSpec((BN, HALF), lambda j: (NB + j, 0)),
            pl.BlockSpec((BN, 1), lambda j: (j, 0)),
            _full((H, H)), _full((H, H)), _full((1, H)), _full((1, H)),
            _full((H, H)), _full((H, H)), _full((1, H)),
        ],
        out_specs=pl.BlockSpec((BN, H), lambda j: (j, 0)),
        out_shape=jax.ShapeDtypeStruct((N, H), _f32),
    )(h, s2, s2, cnt2, u1h, w2r, b2m.reshape(1, H), b1u.reshape(1, H),
      u1a, u2, b2u.reshape(1, H))


def _decode(h, p_dec):
    d2 = jnp.pad(p_dec["W2"], ((0, 0), (0, 128 - p_dec["W2"].shape[1])))
    b2 = jnp.pad(p_dec["b2"], (0, 128 - p_dec["b2"].shape[0])).reshape(1, 128)
    out = pl.pallas_call(
        _dec_body,
        grid=(NB,),
        in_specs=[
            pl.BlockSpec((BN, H), lambda j: (j, 0)),
            _full((H, H)), _full((1, H)), _full((H, 128)), _full((1, 128)),
        ],
        out_specs=pl.BlockSpec((BN, 128), lambda j: (j, 0)),
        out_shape=jax.ShapeDtypeStruct((N, 128), _f32),
    )(h, p_dec["W1"], p_dec["b1"].reshape(1, H), d2, b2)
    return out[:, :3]


ROWS = E // 128
RPT = ROWS // 16

_sc_mesh = plsc.VectorSubcoreMesh(core_axis_name="c", subcore_axis_name="s")


def _sc_edge_body(t_hbm, ce_hbm, src_hbm, dst_hbm, out_hbm,
                  gia, gib, dstb, bufa, bufb, bufc, s_sh,
                  sema, semb, semc, semi):
    c = lax.axis_index("c")
    s = lax.axis_index("s")
    zero16 = jnp.zeros((16,), _f32)

    def _zr(r, _):
        for j in range(8):
            bufa[r, pl.ds(j * 16, 16)] = zero16
        return 0
    lax.fori_loop(0, 104, _zr, 0)
    for r in range(6):
        pltpu.sync_copy(bufa.at[pl.ds(0, 104)],
                        s_sh.at[pl.ds(s * 624 + r * 104, 104)])

    @pl.when(s < 2)
    def _():
        pltpu.sync_copy(bufa.at[pl.ds(0, 8)],
                        s_sh.at[pl.ds(9984 + s * 8, 8)])
    plsc.subcore_barrier()

    off_a = jnp.full((16,), c * N, jnp.int32)
    off_b = jnp.full((16,), 2 * N + c * N, jnp.int32)
    base_row = s * RPT

    def _compute():
        def _rb(r, _):
            for j in range(8):
                sl = pl.ds(j * 16, 16)
                bufc[r, sl] = _irne(jnp.maximum(
                    bufa[r, sl] + bufb[r, sl] + bufc[r, sl], 0.0))
            return 0
        lax.fori_loop(0, 128, _rb, 0)

    pltpu.async_copy(src_hbm.at[pl.ds(base_row, 1)], gia.at[pl.ds(0, 1)], semi)
    pltpu.async_copy(dst_hbm.at[pl.ds(base_row, 1)], dstb.at[pl.ds(0, 1)], semi)

    def _main(i, _):
        p = lax.rem(i, 2)
        row0 = base_row + i
        pltpu.make_async_copy(src_hbm.at[pl.ds(row0, 1)],
                              gia.at[pl.ds(p, 1)], semi).wait()
        pltpu.make_async_copy(dst_hbm.at[pl.ds(row0, 1)],
                              dstb.at[pl.ds(p, 1)], semi).wait()
        @pl.when(i > 0)
        def _():
            pltpu.make_async_copy(bufc, s_sh.at[dstb.at[0]], semc).wait()
        for j in range(8):
            sl = pl.ds(j * 16, 16)
            gia[p, sl] = gia[p, sl] + off_a
            gib[0, sl] = dstb[p, sl] + off_b
        cpa = pltpu.async_copy(t_hbm.at[gia.at[p]], bufa, sema)
        cpb = pltpu.async_copy(t_hbm.at[gib.at[0]], bufb, semb)
        pltpu.async_copy(src_hbm.at[pl.ds(row0 + 1, 1)],
                         gia.at[pl.ds(1 - p, 1)], semi)
        pltpu.async_copy(dst_hbm.at[pl.ds(row0 + 1, 1)],
                         dstb.at[pl.ds(1 - p, 1)], semi)
        pltpu.sync_copy(ce_hbm.at[pl.ds(c * E + row0 * 128, 128)], bufc)
        cpa.wait()
        cpb.wait()
        _compute()
        pltpu.async_copy(bufc, s_sh.at[dstb.at[p]], semc, add=True)
        return 0
    lax.fori_loop(0, RPT, _main, 0)
    pltpu.make_async_copy(src_hbm.at[pl.ds(base_row, 1)],
                          gia.at[pl.ds(0, 1)], semi).wait()
    pltpu.make_async_copy(dst_hbm.at[pl.ds(base_row, 1)],
                          dstb.at[pl.ds(0, 1)], semi).wait()
    pltpu.make_async_copy(bufc, s_sh.at[dstb.at[0]], semc).wait()

    @pl.when(s < 2)
    def _():
        row0 = 16 * RPT + s
        pltpu.sync_copy(src_hbm.at[pl.ds(row0, 1)], gia.at[pl.ds(0, 1)])
        pltpu.sync_copy(dst_hbm.at[pl.ds(row0, 1)], dstb.at[pl.ds(0, 1)])
        for j in range(8):
            sl = pl.ds(j * 16, 16)
            gia[0, sl] = gia[0, sl] + off_a
            gib[0, sl] = dstb[0, sl] + off_b
        cpa = pltpu.async_copy(t_hbm.at[gia.at[0]], bufa, sema)
        cpb = pltpu.async_copy(t_hbm.at[gib.at[0]], bufb, semb)
        pltpu.sync_copy(ce_hbm.at[pl.ds(c * E + row0 * 128, 128)], bufc)
        cpa.wait()
        cpb.wait()
        _compute()
        pltpu.sync_copy(bufc, s_sh.at[dstb.at[0]], add=True)

    plsc.subcore_barrier()
    pltpu.sync_copy(s_sh.at[pl.ds(s * 624, 624)],
                    out_hbm.at[pl.ds(c * N + s * 624, 624)])

    @pl.when(s < 2)
    def _():
        pltpu.sync_copy(s_sh.at[pl.ds(9984 + s * 8, 8)],
                        out_hbm.at[pl.ds(c * N + 9984 + s * 8, 8)])


def _edge_pass(t_tab, ce2, src2d, dst2d):
    return pl.kernel(
        _sc_edge_body,
        out_type=jax.ShapeDtypeStruct((2 * N, HALF), _f32),
        mesh=_sc_mesh,
        scratch_types=[
            pltpu.VMEM((2, 128), jnp.int32),
            pltpu.VMEM((1, 128), jnp.int32),
            pltpu.VMEM((2, 128), jnp.int32),
            pltpu.VMEM((128, HALF), _f32),
            pltpu.VMEM((128, HALF), _f32),
            pltpu.VMEM((128, HALF), _f32),
            pltpu.VMEM_SHARED((N, HALF), _f32),
            pltpu.SemaphoreType.DMA,
            pltpu.SemaphoreType.DMA,
            pltpu.SemaphoreType.DMA,
            pltpu.SemaphoreType.DMA,
        ],
    )(t_tab, ce2, src2d, dst2d)


def _sc_cnt_body(dst_hbm, out_hbm, idx1, vals, cnt_sh, onesb):
    c = lax.axis_index("c")
    s = lax.axis_index("s")
    def _z(i, _):
        vals[pl.ds(i * 16, 16)] = jnp.zeros((16,), _f32)
        return 0
    lax.fori_loop(0, 125, _z, 0)

    @pl.when(s < 5)
    def _():
        pltpu.sync_copy(vals, cnt_sh.at[pl.ds(s * 2000, 2000)])
    plsc.subcore_barrier()

    def _o(i, _):
        onesb[pl.ds(i * 16, 16)] = jnp.ones((16,), _f32)
        return 0
    lax.fori_loop(0, 8, _o, 0)

    wid = c * 16 + s

    def _r(r, _):
        pltpu.sync_copy(dst_hbm.at[pl.ds(wid * 39 + r, 1)], idx1)
        pltpu.sync_copy(onesb, cnt_sh.at[idx1.at[0]], add=True)
        return 0
    lax.fori_loop(0, 39, _r, 0)

    @pl.when(jnp.logical_and(c == 0, s < 2))
    def _():
        pltpu.sync_copy(dst_hbm.at[pl.ds(39 * 32 + s, 1)], idx1)
        pltpu.sync_copy(onesb, cnt_sh.at[idx1.at[0]], add=True)
    plsc.subcore_barrier()

    @pl.when(s < 5)
    def _():
        pltpu.sync_copy(cnt_sh.at[pl.ds(s * 2000, 2000)], vals)
        pltpu.sync_copy(vals, out_hbm.at[pl.ds(c * N + s * 2000, 2000)])


def _edge_counts(dst2d):
    parts = pl.kernel(
        _sc_cnt_body,
        out_type=jax.ShapeDtypeStruct((2 * N,), _f32),
        mesh=_sc_mesh,
        scratch_types=[
            pltpu.VMEM((1, 128), jnp.int32),
            pltpu.VMEM((2000,), _f32),
            pltpu.VMEM_SHARED((N,), _f32),
            pltpu.VMEM((128,), _f32),
        ],
    )(dst2d)
    return parts[:N] + parts[N:]


def kernel(x_t, node_meta, edge_index, edge_feats, t_embed, batch_index, params):
    del batch_index
    src2d = edge_index[0].astype(jnp.int32).reshape(ROWS, 128)
    dst2d = edge_index[1].astype(jnp.int32).reshape(ROWS, 128)

    xm = jnp.pad(jnp.concatenate([x_t, node_meta], axis=1), ((0, 0), (0, 1)))
    ef8 = jnp.pad(edge_feats, ((0, 0), (0, 4)))

    h = _encode_nodes(xm, t_embed, params["enc_node"], params["enc_time"])
    h_edge16 = _encode_edges(ef8, params["enc_edge"])

    cnt2 = _edge_counts(dst2d).reshape(N, 1)

    for blk in params["blocks"]:
        w1 = blk["msg"]["W1"]
        w1s, w1d, w1e = w1[:H], w1[H:2 * H], w1[2 * H:]
        u1 = blk["node"]["W1"]
        u1h, u1a = u1[:H], u1[H:]
        w2r = _irne(blk["msg"]["W2"])

        ce2 = _ce(h_edge16, w1e)
        t_tab = _ab(h, w1s, w1d, blk["msg"]["b1"])
        s2 = _edge_pass(t_tab, ce2, src2d, dst2d)
        h = _node_update(h, s2, cnt2, u1h, w2r, blk["msg"]["b2"],
                         blk["node"]["b1"], u1a, blk["node"]["W2"],
                         blk["node"]["b2"])

    return _decode(h, params["dec_node"])
